# Initial kernel scaffold; baseline (speedup 1.0000x reference)
import jax
import jax.numpy as jnp
from jax.experimental import pallas as pl


def kernel(tokens, W_E):
    raise NotImplementedError("write your pallas kernel here")


# speedup vs baseline: 1.4169x; 1.4169x over previous
import functools

import jax
import jax.numpy as jnp
from jax import lax
from jax.experimental import pallas as pl
from jax.experimental.pallas import tpu as pltpu
from jax.experimental.pallas import tpu_sc as plsc

NC = 2
NS = 16
NW = NC * NS
CH = 64


@functools.lru_cache(maxsize=None)
def _embed_call(B, V, D):
    b_per_w = B // NW
    nchunk = b_per_w // CH
    mesh = plsc.VectorSubcoreMesh(core_axis_name="c", subcore_axis_name="s")

    @functools.partial(
        pl.kernel,
        mesh=mesh,
        out_type=jax.ShapeDtypeStruct((B, D), jnp.float32),
        scratch_types=[
            pltpu.VMEM((b_per_w,), jnp.int32),
            pltpu.VMEM((CH, D), jnp.float32),
            pltpu.SemaphoreType.DMA,
        ],
    )
    def k(tokens_hbm, table_hbm, out_hbm, idx_v, rows_v, sem):
        wid = lax.axis_index("s") * NC + lax.axis_index("c")
        base = wid * b_per_w
        pltpu.sync_copy(tokens_hbm.at[pl.ds(base, b_per_w)], idx_v)
        for c in range(nchunk):
            pltpu.async_copy(
                table_hbm.at[idx_v.at[pl.ds(c * CH, CH)]], rows_v, sem
            ).wait()
            pltpu.sync_copy(rows_v, out_hbm.at[pl.ds(base + c * CH, CH)])

    return k


def kernel(tokens, W_E):
    Bt, S = tokens.shape
    V, D = W_E.shape
    flat = tokens.reshape(-1).astype(jnp.int32)
    out = _embed_call(flat.shape[0], V, D)(flat, W_E)
    return out.reshape(Bt, S, D)


# speedup vs baseline: 1.4756x; 1.0414x over previous
import functools

import jax
import jax.numpy as jnp
from jax import lax
from jax.experimental import pallas as pl
from jax.experimental.pallas import tpu as pltpu
from jax.experimental.pallas import tpu_sc as plsc

NC = 2
NS = 16
NW = NC * NS
CH = 64


@functools.lru_cache(maxsize=None)
def _embed_call(B, V, D):
    b_per_w = B // NW
    nchunk = b_per_w // CH
    mesh = plsc.VectorSubcoreMesh(core_axis_name="c", subcore_axis_name="s")

    @functools.partial(
        pl.kernel,
        mesh=mesh,
        out_type=jax.ShapeDtypeStruct((B, D), jnp.float32),
        scratch_types=[
            pltpu.VMEM((b_per_w,), jnp.int32),
            pltpu.VMEM((CH, D), jnp.float32),
            pltpu.VMEM((CH, D), jnp.float32),
            pltpu.SemaphoreType.DMA,
            pltpu.SemaphoreType.DMA,
            pltpu.SemaphoreType.DMA,
            pltpu.SemaphoreType.DMA,
        ],
    )
    def k(tokens_hbm, table_hbm, out_hbm, idx_v, rows0, rows1, g0, g1, w0, w1):
        wid = lax.axis_index("s") * NC + lax.axis_index("c")
        base = wid * b_per_w
        pltpu.sync_copy(tokens_hbm.at[pl.ds(base, b_per_w)], idx_v)
        bufs, gsems, wsems = (rows0, rows1), (g0, g1), (w0, w1)
        handles = {}

        def start_gather(c):
            handles["g", c] = pltpu.async_copy(
                table_hbm.at[idx_v.at[pl.ds(c * CH, CH)]], bufs[c % 2], gsems[c % 2]
            )

        def start_write(c):
            handles["w", c] = pltpu.async_copy(
                bufs[c % 2], out_hbm.at[pl.ds(base + c * CH, CH)], wsems[c % 2]
            )

        start_gather(0)
        for c in range(nchunk):
            if c >= 1:
                handles["w", c - 1].wait()
            if c + 1 < nchunk:
                start_gather(c + 1)
            handles["g", c].wait()
            start_write(c)
        handles["w", nchunk - 1].wait()

    return k


def kernel(tokens, W_E):
    Bt, S = tokens.shape
    V, D = W_E.shape
    flat = tokens.reshape(-1).astype(jnp.int32)
    out = _embed_call(flat.shape[0], V, D)(flat, W_E)
    return out.reshape(Bt, S, D)


# speedup vs baseline: 1.5198x; 1.0299x over previous
import functools

import jax
import jax.numpy as jnp
from jax import lax
from jax.experimental import pallas as pl
from jax.experimental.pallas import tpu as pltpu
from jax.experimental.pallas import tpu_sc as plsc

NC = 2
NS = 16
NW = NC * NS
CH = 32
NBUF = 4


@functools.lru_cache(maxsize=None)
def _embed_call(Bt, S, V, D):
    B = Bt * S
    b_per_w = B // NW
    nchunk = b_per_w // CH
    w_per_row = S // b_per_w
    mesh = plsc.VectorSubcoreMesh(core_axis_name="c", subcore_axis_name="s")

    @functools.partial(
        pl.kernel,
        mesh=mesh,
        out_type=jax.ShapeDtypeStruct((B, D), jnp.float32),
        scratch_types=[
            pltpu.VMEM((b_per_w,), jnp.int32),
        ]
        + [pltpu.VMEM((CH, D), jnp.float32) for _ in range(NBUF)]
        + [pltpu.SemaphoreType.DMA for _ in range(2 * NBUF)],
    )
    def k(tokens_hbm, table_hbm, out_hbm, idx_v, *bufs_sems):
        bufs = bufs_sems[:NBUF]
        gsems = bufs_sems[NBUF : 2 * NBUF]
        wsems = bufs_sems[2 * NBUF :]
        wid = lax.axis_index("s") * NC + lax.axis_index("c")
        base = wid * b_per_w
        row = wid // w_per_row
        col = (wid % w_per_row) * b_per_w
        pltpu.sync_copy(tokens_hbm.at[row, pl.ds(col, b_per_w)], idx_v)
        handles = {}

        def start_gather(c):
            handles["g", c] = pltpu.async_copy(
                table_hbm.at[idx_v.at[pl.ds(c * CH, CH)]],
                bufs[c % NBUF],
                gsems[c % NBUF],
            )

        def start_write(c):
            handles["w", c] = pltpu.async_copy(
                bufs[c % NBUF],
                out_hbm.at[pl.ds(base + c * CH, CH)],
                wsems[c % NBUF],
            )

        for c in range(min(NBUF - 1, nchunk)):
            start_gather(c)
        for c in range(nchunk):
            nxt = c + NBUF - 1
            if nxt < nchunk:
                if nxt >= NBUF:
                    handles["w", nxt - NBUF].wait()
                start_gather(nxt)
            handles["g", c].wait()
            start_write(c)
        for c in range(max(0, nchunk - NBUF), nchunk):
            handles["w", c].wait()

    return k


def kernel(tokens, W_E):
    Bt, S = tokens.shape
    V, D = W_E.shape
    out = _embed_call(Bt, S, V, D)(tokens, W_E)
    return out.reshape(Bt, S, D)
